# baseline (speedup 1.0000x reference)
import functools

import numpy as np
import jax
import jax.numpy as jnp
from jax import lax
from jax.experimental import pallas as pl
from jax.experimental.pallas import tpu as pltpu
from jax.experimental.pallas import tpu_sc as plsc

_B, _S, _D, _V = 1024, 200, 128, 1000
_NC, _NS = 2, 16
_NW = _NC * _NS
_N = _B * _S
_SENT_PER_W = _B // _NW
_HALF = _S // 2
_NBUF = 3


def _pos_encoding_np():
    pos = np.arange(_S)[:, None].astype(np.float32)
    i = np.arange(_D)[None, :].astype(np.float32)
    angle_rates = 1.0 / np.power(10000.0, (2.0 * np.floor(i / 2.0)) / _D)
    angles = pos * angle_rates
    pe = np.zeros((_S, _D), dtype=np.float32)
    pe[:, 0::2] = np.sin(angles[:, 0::2])
    pe[:, 1::2] = np.cos(angles[:, 1::2])
    return pe


_PE_NP = _pos_encoding_np()

_mesh = plsc.VectorSubcoreMesh(core_axis_name="c", subcore_axis_name="s")


@functools.partial(
    pl.kernel,
    out_type=jax.ShapeDtypeStruct((2 * _B, _HALF, _D), jnp.float32),
    mesh=_mesh,
    scratch_types=[
        pltpu.VMEM_SHARED((_V, _D), jnp.float32),
        pltpu.VMEM((_S, _D), jnp.float32),
        pltpu.VMEM((2 * _SENT_PER_W, _HALF), jnp.int32),
        pltpu.VMEM((_NBUF, _S, _D), jnp.float32),
        pltpu.SemaphoreType.DMA,
        pltpu.SemaphoreType.DMA,
        pltpu.SemaphoreType.DMA,
        pltpu.SemaphoreType.DMA,
        pltpu.SemaphoreType.DMA,
    ],
)
def _emb(table_hbm, x_hbm, pe_hbm, out_hbm, tab_s, pe_v, idx_v, rows_v,
         gsem0, gsem1, isem, osem0, osem1):
    sid = lax.axis_index("s")
    wid = sid * _NC + lax.axis_index("c")
    sent0 = wid * _SENT_PER_W

    @pl.when(sid == 0)
    def _():
        pltpu.sync_copy(table_hbm, tab_s)

    pltpu.async_copy(x_hbm.at[pl.ds(sent0 * 2, 2 * _SENT_PER_W)], idx_v,
                     isem)
    pltpu.sync_copy(pe_hbm, pe_v)
    pltpu.make_async_copy(x_hbm.at[pl.ds(sent0 * 2, 2 * _SENT_PER_W)],
                          idx_v, isem).wait()
    plsc.subcore_barrier()

    def start_gather(sent, slot, sem):
        for h in range(2):
            pltpu.async_copy(
                tab_s.at[idx_v.at[2 * sent + h]],
                rows_v.at[slot, pl.ds(h * _HALF, _HALF)],
                sem)

    def wait_gather(sent, slot, sem):
        for h in range(2):
            pltpu.make_async_copy(
                tab_s.at[idx_v.at[2 * sent + h]],
                rows_v.at[slot, pl.ds(h * _HALF, _HALF)],
                sem).wait()

    start_gather(0, 0, gsem0)
    start_gather(1, 1, gsem1)

    def body(j, carry):
        buf = j % _NBUF

        @pl.when(j % 2 == 0)
        def _():
            wait_gather(j, buf, gsem0)

        @pl.when(j % 2 == 1)
        def _():
            wait_gather(j, buf, gsem1)

        @pl.when(j >= 1)
        def _():
            pltpu.make_async_copy(rows_v.at[0, pl.ds(0, _HALF)],
                                  out_hbm.at[0], osem0).wait()
            pltpu.make_async_copy(rows_v.at[0, pl.ds(0, _HALF)],
                                  out_hbm.at[0], osem1).wait()

        @pl.when(j + 2 < _SENT_PER_W)
        def _():
            nb = (j + 2) % _NBUF

            @pl.when(j % 2 == 0)
            def _():
                start_gather(j + 2, nb, gsem0)

            @pl.when(j % 2 == 1)
            def _():
                start_gather(j + 2, nb, gsem1)

        for h, osem in ((0, osem0), (1, osem1)):
            @plsc.parallel_loop(0, _HALF, unroll=4)
            def _(r):
                for c in range(_D // 16):
                    sl = pl.ds(c * 16, 16)
                    plsc.addupdate(rows_v.at[buf, h * _HALF + r, sl],
                                   pe_v[h * _HALF + r, sl])

            pltpu.async_copy(rows_v.at[buf, pl.ds(h * _HALF, _HALF)],
                             out_hbm.at[(sent0 + j) * 2 + h], osem)
        return carry

    lax.fori_loop(0, _SENT_PER_W, body, 0)
    pltpu.make_async_copy(rows_v.at[0, pl.ds(0, _HALF)], out_hbm.at[0],
                          osem0).wait()
    pltpu.make_async_copy(rows_v.at[0, pl.ds(0, _HALF)], out_hbm.at[0],
                          osem1).wait()


def kernel(x, table):
    xf = x.reshape(_N).astype(jnp.int32).reshape(_N // _HALF, _HALF)
    pe = jnp.asarray(_PE_NP)
    out = _emb(table, xf, pe)
    return out.reshape(_B, _S, _D)


# speedup vs baseline: 1.6975x; 1.6975x over previous
import functools

import numpy as np
import jax
import jax.numpy as jnp
from jax import lax
from jax.experimental import pallas as pl
from jax.experimental.pallas import tpu as pltpu
from jax.experimental.pallas import tpu_sc as plsc

_B, _S, _D, _V = 1024, 200, 128, 1000
_NC, _NS = 2, 16
_NW = _NC * _NS
_N = _B * _S
_SENT_PER_W = _B // _NW
_HALF = _S // 2
_NBUF = 3


def _pos_encoding_np():
    pos = np.arange(_S)[:, None].astype(np.float32)
    i = np.arange(_D)[None, :].astype(np.float32)
    angle_rates = 1.0 / np.power(10000.0, (2.0 * np.floor(i / 2.0)) / _D)
    angles = pos * angle_rates
    pe = np.zeros((_S, _D), dtype=np.float32)
    pe[:, 0::2] = np.sin(angles[:, 0::2])
    pe[:, 1::2] = np.cos(angles[:, 1::2])
    return pe


_PE_NP = _pos_encoding_np()

_mesh = plsc.VectorSubcoreMesh(core_axis_name="c", subcore_axis_name="s")


@functools.partial(
    pl.kernel,
    out_type=jax.ShapeDtypeStruct((_N, _D), jnp.float32),
    mesh=_mesh,
    scratch_types=[
        pltpu.VMEM_SHARED((_V, _D), jnp.float32),
        pltpu.VMEM((_S, _D), jnp.float32),
        pltpu.VMEM((2 * _SENT_PER_W, _HALF), jnp.int32),
        pltpu.VMEM((_NBUF, _S, _D), jnp.float32),
        pltpu.SemaphoreType.DMA,
        pltpu.SemaphoreType.DMA,
        pltpu.SemaphoreType.DMA,
        pltpu.SemaphoreType.DMA,
    ],
)
def _emb(table_hbm, x_hbm, pe_hbm, out_hbm, tab_s, pe_v, idx_v, rows_v,
         gsem0, gsem1, isem, osem):
    sid = lax.axis_index("s")
    wid = sid * _NC + lax.axis_index("c")
    sent0 = wid * _SENT_PER_W

    @pl.when(sid == 0)
    def _():
        pltpu.sync_copy(table_hbm, tab_s)

    pltpu.async_copy(x_hbm.at[pl.ds(sent0 * 2, 2 * _SENT_PER_W)], idx_v,
                     isem)
    pltpu.sync_copy(pe_hbm, pe_v)
    pltpu.make_async_copy(x_hbm.at[pl.ds(sent0 * 2, 2 * _SENT_PER_W)],
                          idx_v, isem).wait()
    plsc.subcore_barrier()

    def start_gather(sent, slot, sem):
        for h in range(2):
            pltpu.async_copy(
                tab_s.at[idx_v.at[2 * sent + h]],
                rows_v.at[slot, pl.ds(h * _HALF, _HALF)],
                sem)

    def wait_gather(sent, slot, sem):
        for h in range(2):
            pltpu.make_async_copy(
                tab_s.at[idx_v.at[2 * sent + h]],
                rows_v.at[slot, pl.ds(h * _HALF, _HALF)],
                sem).wait()

    start_gather(0, 0, gsem0)
    start_gather(1, 1, gsem1)

    def body(j, carry):
        buf = j % _NBUF

        @pl.when(j % 2 == 0)
        def _():
            wait_gather(j, buf, gsem0)

        @pl.when(j % 2 == 1)
        def _():
            wait_gather(j, buf, gsem1)

        @pl.when(j >= 1)
        def _():
            pltpu.make_async_copy(rows_v.at[0], out_hbm.at[pl.ds(0, _S)],
                                  osem).wait()

        @pl.when(j + 2 < _SENT_PER_W)
        def _():
            nb = (j + 2) % _NBUF

            @pl.when(j % 2 == 0)
            def _():
                start_gather(j + 2, nb, gsem0)

            @pl.when(j % 2 == 1)
            def _():
                start_gather(j + 2, nb, gsem1)

        @plsc.parallel_loop(0, _S, unroll=4)
        def _(r):
            for c in range(_D // 16):
                sl = pl.ds(c * 16, 16)
                plsc.addupdate(rows_v.at[buf, r, sl], pe_v[r, sl])

        pltpu.async_copy(rows_v.at[buf],
                         out_hbm.at[pl.ds((sent0 + j) * _S, _S)], osem)
        return carry

    lax.fori_loop(0, _SENT_PER_W, body, 0)
    pltpu.make_async_copy(rows_v.at[0], out_hbm.at[pl.ds(0, _S)],
                          osem).wait()


def kernel(x, table):
    xf = x.reshape(_N).astype(jnp.int32).reshape(_N // _HALF, _HALF)
    pe = jnp.asarray(_PE_NP)
    out = _emb(table, xf, pe)
    return out.reshape(_B, _S, _D)


# speedup vs baseline: 1.7046x; 1.0041x over previous
import functools

import numpy as np
import jax
import jax.numpy as jnp
from jax import lax
from jax.experimental import pallas as pl
from jax.experimental.pallas import tpu as pltpu
from jax.experimental.pallas import tpu_sc as plsc

_B, _S, _D, _V = 1024, 200, 128, 1000
_NC, _NS = 2, 16
_NW = _NC * _NS
_N = _B * _S
_SENT_PER_W = _B // _NW
_HALF = _S // 2
_NBUF = 3


def _pos_encoding_np():
    pos = np.arange(_S)[:, None].astype(np.float32)
    i = np.arange(_D)[None, :].astype(np.float32)
    angle_rates = 1.0 / np.power(10000.0, (2.0 * np.floor(i / 2.0)) / _D)
    angles = pos * angle_rates
    pe = np.zeros((_S, _D), dtype=np.float32)
    pe[:, 0::2] = np.sin(angles[:, 0::2])
    pe[:, 1::2] = np.cos(angles[:, 1::2])
    return pe


_PE_NP = _pos_encoding_np()

_mesh = plsc.VectorSubcoreMesh(core_axis_name="c", subcore_axis_name="s")


@functools.partial(
    pl.kernel,
    out_type=jax.ShapeDtypeStruct((_N, _D), jnp.float32),
    mesh=_mesh,
    scratch_types=[
        pltpu.VMEM_SHARED((_V, _D), jnp.float32),
        pltpu.VMEM((_S, _D), jnp.float32),
        pltpu.VMEM((2 * _SENT_PER_W, _HALF), jnp.int32),
        pltpu.VMEM((_NBUF, _S, _D), jnp.float32),
        pltpu.SemaphoreType.DMA,
        pltpu.SemaphoreType.DMA,
        pltpu.SemaphoreType.DMA,
        pltpu.SemaphoreType.DMA,
    ],
)
def _emb(table_hbm, x_hbm, pe_hbm, out_hbm, tab_s, pe_v, idx_v, rows_v,
         gsem0, gsem1, isem, osem):
    sid = lax.axis_index("s")
    wid = sid * _NC + lax.axis_index("c")
    sent0 = wid * _SENT_PER_W

    @pl.when(sid < 7)
    def _():
        pltpu.sync_copy(table_hbm.at[pl.ds(sid * 128, 128)],
                        tab_s.at[pl.ds(sid * 128, 128)])

    @pl.when(sid == 7)
    def _():
        pltpu.sync_copy(table_hbm.at[pl.ds(896, 104)],
                        tab_s.at[pl.ds(896, 104)])

    pltpu.async_copy(x_hbm.at[pl.ds(sent0 * 2, 2 * _SENT_PER_W)], idx_v,
                     isem)
    pltpu.sync_copy(pe_hbm, pe_v)
    pltpu.make_async_copy(x_hbm.at[pl.ds(sent0 * 2, 2 * _SENT_PER_W)],
                          idx_v, isem).wait()
    plsc.subcore_barrier()

    def start_gather(sent, slot, sem):
        for h in range(2):
            pltpu.async_copy(
                tab_s.at[idx_v.at[2 * sent + h]],
                rows_v.at[slot, pl.ds(h * _HALF, _HALF)],
                sem)

    def wait_gather(sent, slot, sem):
        for h in range(2):
            pltpu.make_async_copy(
                tab_s.at[idx_v.at[2 * sent + h]],
                rows_v.at[slot, pl.ds(h * _HALF, _HALF)],
                sem).wait()

    start_gather(0, 0, gsem0)
    start_gather(1, 1, gsem1)

    def body(j, carry):
        buf = j % _NBUF

        @pl.when(j % 2 == 0)
        def _():
            wait_gather(j, buf, gsem0)

        @pl.when(j % 2 == 1)
        def _():
            wait_gather(j, buf, gsem1)

        @pl.when(j >= 1)
        def _():
            pltpu.make_async_copy(rows_v.at[0], out_hbm.at[pl.ds(0, _S)],
                                  osem).wait()

        @pl.when(j + 2 < _SENT_PER_W)
        def _():
            nb = (j + 2) % _NBUF

            @pl.when(j % 2 == 0)
            def _():
                start_gather(j + 2, nb, gsem0)

            @pl.when(j % 2 == 1)
            def _():
                start_gather(j + 2, nb, gsem1)

        @plsc.parallel_loop(0, _S, unroll=4)
        def _(r):
            for c in range(_D // 16):
                sl = pl.ds(c * 16, 16)
                plsc.addupdate(rows_v.at[buf, r, sl], pe_v[r, sl])

        pltpu.async_copy(rows_v.at[buf],
                         out_hbm.at[pl.ds((sent0 + j) * _S, _S)], osem)
        return carry

    lax.fori_loop(0, _SENT_PER_W, body, 0)
    pltpu.make_async_copy(rows_v.at[0], out_hbm.at[pl.ds(0, _S)],
                          osem).wait()


def kernel(x, table):
    xf = x.reshape(_N).astype(jnp.int32).reshape(_N // _HALF, _HALF)
    pe = jnp.asarray(_PE_NP)
    out = _emb(table, xf, pe)
    return out.reshape(_B, _S, _D)
